# baseline (speedup 1.0000x reference)
import functools

import jax
import jax.numpy as jnp
from jax import lax
from jax.experimental import pallas as pl
from jax.experimental.pallas import tpu as pltpu
from jax.experimental.pallas import tpu_sc as plsc

HIDDEN = 1024
FF = 2816
E = 8
TOKENS = 2048
NA = 2 * TOKENS
TILE_M = 256
N_PAD = NA + E * TILE_M
NT = N_PAD // TILE_M
CH = 512

NW = 32
A_PER_W = NA // NW
T_PER_W = TOKENS // NW
DCH = 64


def _gate_body(x_ref, wg_ref, dest_ref, wflat_ref, te_ref, rows_ref):
    x = x_ref[...]
    wg = wg_ref[...]
    logits = jax.lax.dot_general(
        x, wg, (((1,), (1,)), ((), ())),
        preferred_element_type=jnp.float32,
        precision=jax.lax.Precision.DEFAULT,
    )
    lane = jax.lax.broadcasted_iota(jnp.int32, logits.shape, 1)
    big = jnp.float32(-1e30)
    m0 = jnp.max(logits, axis=1, keepdims=True)
    i0 = jnp.min(jnp.where(logits == m0, lane, E), axis=1, keepdims=True)
    l2 = jnp.where(lane == i0, big, logits)
    m1 = jnp.max(l2, axis=1, keepdims=True)
    i1 = jnp.min(jnp.where(l2 == m1, lane, E), axis=1, keepdims=True)
    e1 = jnp.exp(m1 - m0)
    s0 = 1.0 / (1.0 + e1)
    s1 = e1 / (1.0 + e1)

    e_flat = jnp.concatenate([i0, i1], axis=0)
    wflat_ref[...] = jnp.concatenate([s0, s1], axis=0)
    lane8 = jax.lax.broadcasted_iota(jnp.int32, (NA, E), 1)
    oh = (lane8 == e_flat).astype(jnp.float32)

    r_i = jax.lax.broadcasted_iota(jnp.int32, (CH, CH), 0)
    c_i = jax.lax.broadcasted_iota(jnp.int32, (CH, CH), 1)
    tri = (r_i >= c_i).astype(jnp.float32)
    chunks = []
    carry = jnp.zeros((1, E), jnp.float32)
    for c in range(NA // CH):
        ohc = oh[c * CH:(c + 1) * CH, :]
        cs = jax.lax.dot_general(tri, ohc, (((1,), (0,)), ((), ())),
                                 preferred_element_type=jnp.float32) + carry
        carry = cs[CH - 1:CH, :]
        chunks.append(cs)
    csum = jnp.concatenate(chunks, axis=0)
    counts = carry
    rank = jnp.sum(csum * oh, axis=1, keepdims=True) - 1.0

    pc = jnp.ceil(counts / TILE_M) * TILE_M
    lane_r = jax.lax.broadcasted_iota(jnp.int32, (E, E), 0)
    lane_c = jax.lax.broadcasted_iota(jnp.int32, (E, E), 1)
    pcb = jnp.broadcast_to(pc, (E, E))
    pstart = jnp.sum(jnp.where(lane_r < lane_c, pcb.T, 0.0), axis=0,
                     keepdims=True)
    dest = jnp.sum(oh * pstart, axis=1, keepdims=True) + rank
    dest_ref[...] = dest.astype(jnp.int32)

    tstart = (jax.lax.broadcasted_iota(jnp.int32, (NT, 1), 0)
              ).astype(jnp.float32) * TILE_M
    psb = jnp.broadcast_to(pstart, (NT, E))
    te = jnp.sum((psb <= tstart).astype(jnp.int32), axis=1, keepdims=True) - 1
    teoh = (jax.lax.broadcasted_iota(jnp.int32, (NT, E), 1) == te
            ).astype(jnp.float32)
    cnt_t = jnp.sum(teoh * counts, axis=1, keepdims=True)
    ps_t = jnp.sum(teoh * pstart, axis=1, keepdims=True)
    rows = jnp.clip(cnt_t - (tstart - ps_t), 0.0, float(TILE_M))
    te_ref[...] = te
    rows_ref[...] = rows.astype(jnp.int32)


def _dispatch_body(h_ref, dest_ref, wflat_ref, xs_ref, wslot_ref,
                   idx_v, w_v, rows_v, sem):
    wid = lax.axis_index("c") * 16 + lax.axis_index("s")
    base = wid * A_PER_W
    for c in range(A_PER_W // DCH):
        off = base + c * DCH
        pltpu.sync_copy(dest_ref.at[pl.ds(off, DCH)], idx_v)
        pltpu.sync_copy(wflat_ref.at[pl.ds(off, DCH)], w_v)
        tok = lax.rem(off, TOKENS)
        pltpu.sync_copy(h_ref.at[pl.ds(tok, DCH)], rows_v)
        pltpu.async_copy(rows_v, xs_ref.at[idx_v], sem).wait()
        pltpu.async_copy(w_v, wslot_ref.at[idx_v], sem).wait()


CC = 32


def _combine_body(ys_ref, d0_ref, d1_ref, y_ref, i0_v, i1_v, b0_v, b1_v, sem):
    wid = lax.axis_index("c") * 16 + lax.axis_index("s")
    base = wid * T_PER_W
    for c in range(T_PER_W // CC):
        off = base + c * CC
        pltpu.sync_copy(d0_ref.at[pl.ds(off, CC)], i0_v)
        pltpu.sync_copy(d1_ref.at[pl.ds(off, CC)], i1_v)
        cp0 = pltpu.async_copy(ys_ref.at[i0_v], b0_v, sem)
        cp1 = pltpu.async_copy(ys_ref.at[i1_v], b1_v, sem)
        cp0.wait()
        cp1.wait()

        def _row(r, _):
            for j in range(HIDDEN // 16):
                sl = pl.ds(j * 16, 16)
                b0_v[r, sl] = b0_v[r, sl] + b1_v[r, sl]
            return _

        lax.fori_loop(0, CC, _row, 0)
        pltpu.sync_copy(b0_v, y_ref.at[pl.ds(off, CC)])


def _ffn_body(te_ref, rows_ref, xs_ref, w1_ref, w2_ref, wt_ref, o_ref):
    i = pl.program_id(0)
    rows = rows_ref[i, 0]

    @pl.when(rows > 0)
    def _():
        xb = xs_ref[...].astype(jnp.bfloat16)
        w1 = w1_ref[0].astype(jnp.bfloat16)
        w2 = w2_ref[0].astype(jnp.bfloat16)
        z = jax.lax.dot_general(xb, w1, (((1,), (1,)), ((), ())),
                                preferred_element_type=jnp.float32)
        h = z * jax.nn.sigmoid(z)
        y = jax.lax.dot_general(h.astype(jnp.bfloat16), w2,
                                (((1,), (1,)), ((), ())),
                                preferred_element_type=jnp.float32)
        o_ref[...] = y * wt_ref[...]


@functools.lru_cache(maxsize=None)
def _sc_kernels():
    mesh = plsc.VectorSubcoreMesh(core_axis_name="c", subcore_axis_name="s")
    dispatch = pl.kernel(
        _dispatch_body,
        out_type=(
            jax.ShapeDtypeStruct((N_PAD, HIDDEN), jnp.float32),
            jax.ShapeDtypeStruct((N_PAD,), jnp.float32),
        ),
        mesh=mesh,
        scratch_types=[
            pltpu.VMEM((DCH,), jnp.int32),
            pltpu.VMEM((DCH,), jnp.float32),
            pltpu.VMEM((DCH, HIDDEN), jnp.float32),
            pltpu.SemaphoreType.DMA,
        ],
    )
    combine = pl.kernel(
        _combine_body,
        out_type=jax.ShapeDtypeStruct((TOKENS, HIDDEN), jnp.float32),
        mesh=mesh,
        scratch_types=[
            pltpu.VMEM((CC,), jnp.int32),
            pltpu.VMEM((CC,), jnp.int32),
            pltpu.VMEM((CC, HIDDEN), jnp.float32),
            pltpu.VMEM((CC, HIDDEN), jnp.float32),
            pltpu.SemaphoreType.DMA,
        ],
    )
    return dispatch, combine


@jax.jit
def kernel(x, Wg, W1, W2):
    b, t, d = x.shape
    h = x.reshape(t, d)

    dest, wflat, te, rows_active = pl.pallas_call(
        _gate_body,
        out_shape=(
            jax.ShapeDtypeStruct((NA, 1), jnp.int32),
            jax.ShapeDtypeStruct((NA, 1), jnp.float32),
            jax.ShapeDtypeStruct((NT, 1), jnp.int32),
            jax.ShapeDtypeStruct((NT, 1), jnp.int32),
        ),
    )(h, Wg)

    dest1 = dest.reshape(NA)
    _dispatch, _combine = _sc_kernels()
    xs, wslot = _dispatch(h, dest1, wflat.reshape(NA))

    ys = pl.pallas_call(
        _ffn_body,
        grid_spec=pltpu.PrefetchScalarGridSpec(
            num_scalar_prefetch=2,
            grid=(NT,),
            in_specs=[
                pl.BlockSpec((TILE_M, HIDDEN), lambda i, te, ra: (i, 0)),
                pl.BlockSpec((1, FF, HIDDEN), lambda i, te, ra: (te[i, 0], 0, 0)),
                pl.BlockSpec((1, HIDDEN, FF), lambda i, te, ra: (te[i, 0], 0, 0)),
                pl.BlockSpec((TILE_M, 1), lambda i, te, ra: (i, 0)),
            ],
            out_specs=pl.BlockSpec((TILE_M, HIDDEN), lambda i, te, ra: (i, 0)),
        ),
        out_shape=jax.ShapeDtypeStruct((N_PAD, HIDDEN), jnp.float32),
    )(te, rows_active, xs, W1, W2, wslot.reshape(N_PAD, 1))

    y = _combine(ys, dest1[:TOKENS], dest1[TOKENS:])
    return y.reshape(b, t, d)


# speedup vs baseline: 1.0502x; 1.0502x over previous
import functools

import jax
import jax.numpy as jnp
from jax import lax
from jax.experimental import pallas as pl
from jax.experimental.pallas import tpu as pltpu
from jax.experimental.pallas import tpu_sc as plsc

HIDDEN = 1024
FF = 2816
E = 8
TOKENS = 2048
NA = 2 * TOKENS
TILE_M = 256
N_PAD = NA + E * TILE_M
NT = N_PAD // TILE_M
CH = 512

NW = 32
A_PER_W = NA // NW
T_PER_W = TOKENS // NW
DCH = 64


def _gate_body(x_ref, wg_ref, dest_ref, wflat_ref, te_ref, rows_ref):
    x = x_ref[...]
    wg = wg_ref[...]
    logits = jax.lax.dot_general(
        x, wg, (((1,), (1,)), ((), ())),
        preferred_element_type=jnp.float32,
        precision=jax.lax.Precision.DEFAULT,
    )
    lane = jax.lax.broadcasted_iota(jnp.int32, logits.shape, 1)
    big = jnp.float32(-1e30)
    m0 = jnp.max(logits, axis=1, keepdims=True)
    i0 = jnp.min(jnp.where(logits == m0, lane, E), axis=1, keepdims=True)
    l2 = jnp.where(lane == i0, big, logits)
    m1 = jnp.max(l2, axis=1, keepdims=True)
    i1 = jnp.min(jnp.where(l2 == m1, lane, E), axis=1, keepdims=True)
    e1 = jnp.exp(m1 - m0)
    s0 = 1.0 / (1.0 + e1)
    s1 = e1 / (1.0 + e1)

    e_flat = jnp.concatenate([i0, i1], axis=0)
    wflat_ref[...] = jnp.concatenate([s0, s1], axis=0)
    lane8 = jax.lax.broadcasted_iota(jnp.int32, (NA, E), 1)
    oh = (lane8 == e_flat).astype(jnp.float32)

    r_i = jax.lax.broadcasted_iota(jnp.int32, (CH, CH), 0)
    c_i = jax.lax.broadcasted_iota(jnp.int32, (CH, CH), 1)
    tri = (r_i >= c_i).astype(jnp.float32)
    chunks = []
    carry = jnp.zeros((1, E), jnp.float32)
    for c in range(NA // CH):
        ohc = oh[c * CH:(c + 1) * CH, :]
        cs = jax.lax.dot_general(tri, ohc, (((1,), (0,)), ((), ())),
                                 preferred_element_type=jnp.float32) + carry
        carry = cs[CH - 1:CH, :]
        chunks.append(cs)
    csum = jnp.concatenate(chunks, axis=0)
    counts = carry
    rank = jnp.sum(csum * oh, axis=1, keepdims=True) - 1.0

    pc = jnp.ceil(counts / TILE_M) * TILE_M
    lane_r = jax.lax.broadcasted_iota(jnp.int32, (E, E), 0)
    lane_c = jax.lax.broadcasted_iota(jnp.int32, (E, E), 1)
    pcb = jnp.broadcast_to(pc, (E, E))
    pstart = jnp.sum(jnp.where(lane_r < lane_c, pcb.T, 0.0), axis=0,
                     keepdims=True)
    dest = jnp.sum(oh * pstart, axis=1, keepdims=True) + rank
    dest_ref[...] = dest.astype(jnp.int32)

    tstart = (jax.lax.broadcasted_iota(jnp.int32, (NT, 1), 0)
              ).astype(jnp.float32) * TILE_M
    psb = jnp.broadcast_to(pstart, (NT, E))
    te = jnp.sum((psb <= tstart).astype(jnp.int32), axis=1, keepdims=True) - 1
    teoh = (jax.lax.broadcasted_iota(jnp.int32, (NT, E), 1) == te
            ).astype(jnp.float32)
    cnt_t = jnp.sum(teoh * counts, axis=1, keepdims=True)
    ps_t = jnp.sum(teoh * pstart, axis=1, keepdims=True)
    rows = jnp.clip(cnt_t - (tstart - ps_t), 0.0, float(TILE_M))
    te_ref[...] = te
    rows_ref[...] = rows.astype(jnp.int32)


def _dispatch_body(h_ref, dest_ref, wflat_ref, xs_ref, wslot_ref,
                   idx_v, w_v, rows_v, sem):
    wid = lax.axis_index("c") * 16 + lax.axis_index("s")
    base = wid * A_PER_W
    for c in range(A_PER_W // DCH):
        off = base + c * DCH
        pltpu.sync_copy(dest_ref.at[pl.ds(off, DCH)], idx_v)
        pltpu.sync_copy(wflat_ref.at[pl.ds(off, DCH)], w_v)
        tok = lax.rem(off, TOKENS)
        pltpu.sync_copy(h_ref.at[pl.ds(tok, DCH)], rows_v)
        pltpu.async_copy(rows_v, xs_ref.at[idx_v], sem).wait()
        pltpu.async_copy(w_v, wslot_ref.at[idx_v], sem).wait()


CC = 32


def _combine_body(ys_ref, d0_ref, d1_ref, y_ref, i0_v, i1_v, b0_v, b1_v, sem):
    wid = lax.axis_index("c") * 16 + lax.axis_index("s")
    base = wid * T_PER_W
    for c in range(T_PER_W // CC):
        off = base + c * CC
        pltpu.sync_copy(d0_ref.at[pl.ds(off, CC)], i0_v)
        pltpu.sync_copy(d1_ref.at[pl.ds(off, CC)], i1_v)
        cp0 = pltpu.async_copy(ys_ref.at[i0_v], b0_v, sem)
        cp1 = pltpu.async_copy(ys_ref.at[i1_v], b1_v, sem)
        cp0.wait()
        cp1.wait()

        def _row(r, _):
            for j in range(HIDDEN // 16):
                sl = pl.ds(j * 16, 16)
                b0_v[r, sl] = b0_v[r, sl] + b1_v[r, sl]
            return _

        lax.fori_loop(0, CC, _row, 0)
        pltpu.sync_copy(b0_v, y_ref.at[pl.ds(off, CC)])


def _ffn_body(te_ref, rows_ref, xs_ref, w1_ref, w2_ref, wt_ref, o_ref):
    i = pl.program_id(0)
    rows = rows_ref[i, 0]

    @pl.when(rows > 0)
    def _():
        xi = xs_ref[...]
        lo = jax.lax.bitcast_convert_type(
            jax.lax.shift_left(xi, 16), jnp.float32)
        hi = jax.lax.bitcast_convert_type(
            jax.lax.bitwise_and(xi, jnp.int32(-65536)), jnp.float32)
        xb = jnp.concatenate([lo, hi], axis=1).astype(jnp.bfloat16)
        w1 = w1_ref[0].astype(jnp.bfloat16)
        w2 = w2_ref[0].astype(jnp.bfloat16)
        z = jax.lax.dot_general(xb, w1, (((1,), (1,)), ((), ())),
                                preferred_element_type=jnp.float32)
        h = z * jax.nn.sigmoid(z)
        y = jax.lax.dot_general(h.astype(jnp.bfloat16), w2,
                                (((1,), (1,)), ((), ())),
                                preferred_element_type=jnp.float32)
        o_ref[...] = y * wt_ref[...]


@functools.lru_cache(maxsize=None)
def _sc_kernels():
    mesh = plsc.VectorSubcoreMesh(core_axis_name="c", subcore_axis_name="s")
    dispatch = pl.kernel(
        _dispatch_body,
        out_type=(
            jax.ShapeDtypeStruct((N_PAD, HIDDEN // 2), jnp.int32),
            jax.ShapeDtypeStruct((N_PAD,), jnp.float32),
        ),
        mesh=mesh,
        scratch_types=[
            pltpu.VMEM((DCH,), jnp.int32),
            pltpu.VMEM((DCH,), jnp.float32),
            pltpu.VMEM((DCH, HIDDEN // 2), jnp.int32),
            pltpu.SemaphoreType.DMA,
        ],
    )
    combine = pl.kernel(
        _combine_body,
        out_type=jax.ShapeDtypeStruct((TOKENS, HIDDEN), jnp.float32),
        mesh=mesh,
        scratch_types=[
            pltpu.VMEM((CC,), jnp.int32),
            pltpu.VMEM((CC,), jnp.int32),
            pltpu.VMEM((CC, HIDDEN), jnp.float32),
            pltpu.VMEM((CC, HIDDEN), jnp.float32),
            pltpu.SemaphoreType.DMA,
        ],
    )
    return dispatch, combine


@jax.jit
def kernel(x, Wg, W1, W2):
    b, t, d = x.shape
    h = x.reshape(t, d)

    dest, wflat, te, rows_active = pl.pallas_call(
        _gate_body,
        out_shape=(
            jax.ShapeDtypeStruct((NA, 1), jnp.int32),
            jax.ShapeDtypeStruct((NA, 1), jnp.float32),
            jax.ShapeDtypeStruct((NT, 1), jnp.int32),
            jax.ShapeDtypeStruct((NT, 1), jnp.int32),
        ),
    )(h, Wg)

    dest1 = dest.reshape(NA)
    _dispatch, _combine = _sc_kernels()
    hb = h.astype(jnp.bfloat16)
    h32 = jax.lax.bitcast_convert_type(
        jnp.stack([hb[:, :HIDDEN // 2], hb[:, HIDDEN // 2:]], axis=-1),
        jnp.int32)
    xs, wslot = _dispatch(h32, dest1, wflat.reshape(NA))

    ys = pl.pallas_call(
        _ffn_body,
        grid_spec=pltpu.PrefetchScalarGridSpec(
            num_scalar_prefetch=2,
            grid=(NT,),
            in_specs=[
                pl.BlockSpec((TILE_M, HIDDEN // 2), lambda i, te, ra: (i, 0)),
                pl.BlockSpec((1, FF, HIDDEN), lambda i, te, ra: (te[i, 0], 0, 0)),
                pl.BlockSpec((1, HIDDEN, FF), lambda i, te, ra: (te[i, 0], 0, 0)),
                pl.BlockSpec((TILE_M, 1), lambda i, te, ra: (i, 0)),
            ],
            out_specs=pl.BlockSpec((TILE_M, HIDDEN), lambda i, te, ra: (i, 0)),
        ),
        out_shape=jax.ShapeDtypeStruct((N_PAD, HIDDEN), jnp.float32),
    )(te, rows_active, xs, W1, W2, wslot.reshape(N_PAD, 1))

    y = _combine(ys, dest1[:TOKENS], dest1[TOKENS:])
    return y.reshape(b, t, d)


# speedup vs baseline: 1.1352x; 1.0810x over previous
import functools

import jax
import jax.numpy as jnp
from jax import lax
from jax.experimental import pallas as pl
from jax.experimental.pallas import tpu as pltpu
from jax.experimental.pallas import tpu_sc as plsc

HIDDEN = 1024
FF = 2816
E = 8
TOKENS = 2048
NA = 2 * TOKENS
TILE_M = 512
N_PAD = NA + E * TILE_M
NT = N_PAD // TILE_M
CH = 512

NW = 32
A_PER_W = NA // NW
T_PER_W = TOKENS // NW
DCH = 64


def _gate_body(x_ref, wg_ref, dest_ref, wflat_ref, te_ref, rows_ref):
    x = x_ref[...]
    wg = wg_ref[...]
    logits = jax.lax.dot_general(
        x, wg, (((1,), (1,)), ((), ())),
        preferred_element_type=jnp.float32,
        precision=jax.lax.Precision.DEFAULT,
    )
    lane = jax.lax.broadcasted_iota(jnp.int32, logits.shape, 1)
    big = jnp.float32(-1e30)
    m0 = jnp.max(logits, axis=1, keepdims=True)
    i0 = jnp.min(jnp.where(logits == m0, lane, E), axis=1, keepdims=True)
    l2 = jnp.where(lane == i0, big, logits)
    m1 = jnp.max(l2, axis=1, keepdims=True)
    i1 = jnp.min(jnp.where(l2 == m1, lane, E), axis=1, keepdims=True)
    e1 = jnp.exp(m1 - m0)
    s0 = 1.0 / (1.0 + e1)
    s1 = e1 / (1.0 + e1)

    e_flat = jnp.concatenate([i0, i1], axis=0)
    wflat_ref[...] = jnp.concatenate([s0, s1], axis=0)
    lane8 = jax.lax.broadcasted_iota(jnp.int32, (NA, E), 1)
    oh = (lane8 == e_flat).astype(jnp.float32)

    r_i = jax.lax.broadcasted_iota(jnp.int32, (CH, CH), 0)
    c_i = jax.lax.broadcasted_iota(jnp.int32, (CH, CH), 1)
    tri = (r_i >= c_i).astype(jnp.float32)
    chunks = []
    carry = jnp.zeros((1, E), jnp.float32)
    for c in range(NA // CH):
        ohc = oh[c * CH:(c + 1) * CH, :]
        cs = jax.lax.dot_general(tri, ohc, (((1,), (0,)), ((), ())),
                                 preferred_element_type=jnp.float32) + carry
        carry = cs[CH - 1:CH, :]
        chunks.append(cs)
    csum = jnp.concatenate(chunks, axis=0)
    counts = carry
    rank = jnp.sum(csum * oh, axis=1, keepdims=True) - 1.0

    pc = jnp.ceil(counts / TILE_M) * TILE_M
    lane_r = jax.lax.broadcasted_iota(jnp.int32, (E, E), 0)
    lane_c = jax.lax.broadcasted_iota(jnp.int32, (E, E), 1)
    pcb = jnp.broadcast_to(pc, (E, E))
    pstart = jnp.sum(jnp.where(lane_r < lane_c, pcb.T, 0.0), axis=0,
                     keepdims=True)
    dest = jnp.sum(oh * pstart, axis=1, keepdims=True) + rank
    dest_ref[...] = dest.astype(jnp.int32)

    tstart = (jax.lax.broadcasted_iota(jnp.int32, (NT, 1), 0)
              ).astype(jnp.float32) * TILE_M
    psb = jnp.broadcast_to(pstart, (NT, E))
    te = jnp.sum((psb <= tstart).astype(jnp.int32), axis=1, keepdims=True) - 1
    teoh = (jax.lax.broadcasted_iota(jnp.int32, (NT, E), 1) == te
            ).astype(jnp.float32)
    cnt_t = jnp.sum(teoh * counts, axis=1, keepdims=True)
    ps_t = jnp.sum(teoh * pstart, axis=1, keepdims=True)
    rows = jnp.clip(cnt_t - (tstart - ps_t), 0.0, float(TILE_M))
    te_ref[...] = te
    rows_ref[...] = rows.astype(jnp.int32)


def _dispatch_body(h_ref, dest_ref, wflat_ref, xs_ref, wslot_ref,
                   idx_v, w_v, rows_v, sem):
    wid = lax.axis_index("c") * 16 + lax.axis_index("s")
    base = wid * A_PER_W
    for c in range(A_PER_W // DCH):
        off = base + c * DCH
        pltpu.sync_copy(dest_ref.at[pl.ds(off, DCH)], idx_v)
        pltpu.sync_copy(wflat_ref.at[pl.ds(off, DCH)], w_v)
        tok = lax.rem(off, TOKENS)
        pltpu.sync_copy(h_ref.at[pl.ds(tok, DCH)], rows_v)
        pltpu.async_copy(rows_v, xs_ref.at[idx_v], sem).wait()
        pltpu.async_copy(w_v, wslot_ref.at[idx_v], sem).wait()


CC = 32


def _combine_body(ys_ref, d0_ref, d1_ref, y_ref, i0_v, i1_v, b0_v, b1_v, sem):
    wid = lax.axis_index("c") * 16 + lax.axis_index("s")
    base = wid * T_PER_W
    for c in range(T_PER_W // CC):
        off = base + c * CC
        pltpu.sync_copy(d0_ref.at[pl.ds(off, CC)], i0_v)
        pltpu.sync_copy(d1_ref.at[pl.ds(off, CC)], i1_v)
        cp0 = pltpu.async_copy(ys_ref.at[i0_v], b0_v, sem)
        cp1 = pltpu.async_copy(ys_ref.at[i1_v], b1_v, sem)
        cp0.wait()
        cp1.wait()

        def _row(r, _):
            for j in range(HIDDEN // 16):
                sl = pl.ds(j * 16, 16)
                b0_v[r, sl] = b0_v[r, sl] + b1_v[r, sl]
            return _

        lax.fori_loop(0, CC, _row, 0)
        pltpu.sync_copy(b0_v, y_ref.at[pl.ds(off, CC)])


def _ffn_body(te_ref, rows_ref, xs_ref, w1_ref, w2_ref, wt_ref, o_ref):
    i = pl.program_id(0)
    rows = rows_ref[i, 0]

    @pl.when(rows > 0)
    def _():
        xi = xs_ref[...]
        lo = jax.lax.bitcast_convert_type(
            jax.lax.shift_left(xi, 16), jnp.float32)
        hi = jax.lax.bitcast_convert_type(
            jax.lax.bitwise_and(xi, jnp.int32(-65536)), jnp.float32)
        xb = jnp.concatenate([lo, hi], axis=1).astype(jnp.bfloat16)
        w1 = w1_ref[0].astype(jnp.bfloat16)
        w2 = w2_ref[0].astype(jnp.bfloat16)
        z = jax.lax.dot_general(xb, w1, (((1,), (1,)), ((), ())),
                                preferred_element_type=jnp.float32)
        h = z * jax.nn.sigmoid(z)
        y = jax.lax.dot_general(h.astype(jnp.bfloat16), w2,
                                (((1,), (1,)), ((), ())),
                                preferred_element_type=jnp.float32)
        o_ref[...] = y * wt_ref[...]


@functools.lru_cache(maxsize=None)
def _sc_kernels():
    mesh = plsc.VectorSubcoreMesh(core_axis_name="c", subcore_axis_name="s")
    dispatch = pl.kernel(
        _dispatch_body,
        out_type=(
            jax.ShapeDtypeStruct((N_PAD, HIDDEN // 2), jnp.int32),
            jax.ShapeDtypeStruct((N_PAD,), jnp.float32),
        ),
        mesh=mesh,
        scratch_types=[
            pltpu.VMEM((DCH,), jnp.int32),
            pltpu.VMEM((DCH,), jnp.float32),
            pltpu.VMEM((DCH, HIDDEN // 2), jnp.int32),
            pltpu.SemaphoreType.DMA,
        ],
    )
    combine = pl.kernel(
        _combine_body,
        out_type=jax.ShapeDtypeStruct((TOKENS, HIDDEN), jnp.float32),
        mesh=mesh,
        scratch_types=[
            pltpu.VMEM((CC,), jnp.int32),
            pltpu.VMEM((CC,), jnp.int32),
            pltpu.VMEM((CC, HIDDEN), jnp.float32),
            pltpu.VMEM((CC, HIDDEN), jnp.float32),
            pltpu.SemaphoreType.DMA,
        ],
    )
    return dispatch, combine


@jax.jit
def kernel(x, Wg, W1, W2):
    b, t, d = x.shape
    h = x.reshape(t, d)

    dest, wflat, te, rows_active = pl.pallas_call(
        _gate_body,
        out_shape=(
            jax.ShapeDtypeStruct((NA, 1), jnp.int32),
            jax.ShapeDtypeStruct((NA, 1), jnp.float32),
            jax.ShapeDtypeStruct((NT, 1), jnp.int32),
            jax.ShapeDtypeStruct((NT, 1), jnp.int32),
        ),
    )(h, Wg)

    dest1 = dest.reshape(NA)
    _dispatch, _combine = _sc_kernels()
    hb = h.astype(jnp.bfloat16)
    h32 = jax.lax.bitcast_convert_type(
        jnp.stack([hb[:, :HIDDEN // 2], hb[:, HIDDEN // 2:]], axis=-1),
        jnp.int32)
    xs, wslot = _dispatch(h32, dest1, wflat.reshape(NA))

    ys = pl.pallas_call(
        _ffn_body,
        grid_spec=pltpu.PrefetchScalarGridSpec(
            num_scalar_prefetch=2,
            grid=(NT,),
            in_specs=[
                pl.BlockSpec((TILE_M, HIDDEN // 2), lambda i, te, ra: (i, 0)),
                pl.BlockSpec((1, FF, HIDDEN), lambda i, te, ra: (te[i, 0], 0, 0)),
                pl.BlockSpec((1, HIDDEN, FF), lambda i, te, ra: (te[i, 0], 0, 0)),
                pl.BlockSpec((TILE_M, 1), lambda i, te, ra: (i, 0)),
            ],
            out_specs=pl.BlockSpec((TILE_M, HIDDEN), lambda i, te, ra: (i, 0)),
        ),
        out_shape=jax.ShapeDtypeStruct((N_PAD, HIDDEN), jnp.float32),
    )(te, rows_active, xs, W1, W2, wslot.reshape(N_PAD, 1))

    y = _combine(ys, dest1[:TOKENS], dest1[TOKENS:])
    return y.reshape(b, t, d)
